# Initial kernel scaffold; baseline (speedup 1.0000x reference)
import jax
import jax.numpy as jnp
from jax.experimental import pallas as pl


def kernel(indices, table):
    raise NotImplementedError("write your pallas kernel here")


# speedup vs baseline: 2.7994x; 2.7994x over previous
import functools

import jax
import jax.numpy as jnp
from jax import lax
from jax.experimental import pallas as pl
from jax.experimental.pallas import tpu as pltpu
from jax.experimental.pallas import tpu_sc as plsc

VOCAB = 1000000
DIM = 32
BATCH = 16384
HIST = 50

NC = 2
NS = 16
NW = NC * NS
BW = BATCH // NW
C = 64
NCHUNK = BW // C
ROWS = C * HIST
SUB = 128
NSUB = ROWS // SUB


def _make_kernel():
  mesh = plsc.VectorSubcoreMesh(core_axis_name="c", subcore_axis_name="s")

  @functools.partial(
      pl.kernel,
      mesh=mesh,
      out_type=jax.ShapeDtypeStruct((BATCH, DIM), jnp.float32),
      scratch_types=[
          pltpu.VMEM((NSUB, SUB), jnp.int32),
          pltpu.VMEM((ROWS, DIM), jnp.float32),
          pltpu.VMEM((C, DIM), jnp.float32),
          pltpu.SemaphoreType.DMA,
      ],
      compiler_params=pltpu.CompilerParams(use_tc_tiling_on_sc=False),
  )
  def emb_bag(table_hbm, idx_hbm, out_hbm, idx_v, rows_v, out_v, sem):
    wid = lax.axis_index("s") * NC + lax.axis_index("c")
    scale = jnp.float32(1.0 / HIST)

    def chunk_body(c, _):
      pltpu.sync_copy(idx_hbm.at[wid, c], idx_v)

      def fire(s, _):
        pltpu.async_copy(
            table_hbm.at[idx_v.at[s]],
            rows_v.at[pl.ds(s * SUB, SUB)],
            sem,
        )
        return 0

      lax.fori_loop(0, NSUB, fire, 0)
      pltpu.make_async_copy(table_hbm.at[pl.ds(0, ROWS)], rows_v, sem).wait()

      def bag_body(j, _):
        base = j * HIST
        acc0 = rows_v[base, pl.ds(0, 16)]
        acc1 = rows_v[base, pl.ds(16, 16)]
        for l in range(1, HIST):
          acc0 = acc0 + rows_v[base + l, pl.ds(0, 16)]
          acc1 = acc1 + rows_v[base + l, pl.ds(16, 16)]
        out_v[j, pl.ds(0, 16)] = acc0 * scale
        out_v[j, pl.ds(16, 16)] = acc1 * scale
        return 0

      lax.fori_loop(0, C, bag_body, 0)

      pltpu.sync_copy(out_v, out_hbm.at[pl.ds(wid * BW + c * C, C)])
      return 0

    lax.fori_loop(0, NCHUNK, chunk_body, 0)

  return emb_bag


_emb_bag = _make_kernel()


@jax.jit
def kernel(indices, table):
  idx4 = indices.reshape(NW, NCHUNK, NSUB, SUB).astype(jnp.int32)
  return _emb_bag(table, idx4)
